# baseline (speedup 1.0000x reference)
import functools

import jax
import jax.numpy as jnp
from jax import lax
from jax.experimental import pallas as pl
from jax.experimental.pallas import tpu as pltpu
from jax.experimental.pallas import tpu_sc as plsc

N_NODES = 10000
N_EDGES = 320000
D = 128

NC = 2
NS = 16
CHUNK = 128
CHUNKS_PER_TILE = 80
IDX_HALF = 40
NBUF = 2
E_TILE = CHUNK * CHUNKS_PER_TILE
E_PAD = E_TILE * NC * NS
N_PAD = 10112
ROWS_PER_TILE = N_PAD // NS


def _sc_aggregate(x, src_p, dst_p):
  mesh = plsc.VectorSubcoreMesh(core_axis_name="c", subcore_axis_name="s")

  @functools.partial(
      pl.kernel,
      mesh=mesh,
      out_type=jax.ShapeDtypeStruct((NC, N_PAD, D), jnp.float32),
      scratch_types=[
          pltpu.VMEM((IDX_HALF, CHUNK), jnp.int32),
          pltpu.VMEM((IDX_HALF, CHUNK), jnp.int32),
          pltpu.VMEM((NBUF, CHUNK, D), jnp.float32),
          pltpu.VMEM_SHARED((N_PAD, D), jnp.float32),
          pltpu.SemaphoreType.DMA,
          pltpu.SemaphoreType.DMA,
      ],
  )
  def agg_kernel(x_hbm, src_hbm, dst_hbm, out_hbm, sidx_v, didx_v, rows_v, acc,
                 gsem, ssem):
    c = lax.axis_index("c")
    s = lax.axis_index("s")
    w = s * NC + c

    def zero_row(i, _):
      for j in range(D // 16):
        rows_v[0, i, pl.ds(j * 16, 16)] = jnp.zeros((16,), jnp.float32)
      return 0
    lax.fori_loop(0, CHUNK, zero_row, 0)
    full = ROWS_PER_TILE // CHUNK
    for k in range(full):
      pltpu.sync_copy(rows_v.at[0],
                      acc.at[pl.ds(s * ROWS_PER_TILE + k * CHUNK, CHUNK)])
    rem = ROWS_PER_TILE - full * CHUNK
    if rem:
      pltpu.sync_copy(
          rows_v.at[0, pl.ds(0, rem)],
          acc.at[pl.ds(s * ROWS_PER_TILE + full * CHUNK, rem)])
    plsc.subcore_barrier()

    def gather(j, b):
      return pltpu.make_async_copy(x_hbm.at[sidx_v.at[j]], rows_v.at[b], gsem)

    def scatter(j, b):
      return pltpu.async_copy(rows_v.at[b], acc.at[didx_v.at[j]], ssem,
                              add=True)

    def scatter_wait(j, b):
      pltpu.make_async_copy(rows_v.at[b], acc.at[didx_v.at[j]], ssem).wait()

    for h in range(CHUNKS_PER_TILE // IDX_HALF):
      pltpu.sync_copy(
          src_hbm.at[pl.ds(w * CHUNKS_PER_TILE + h * IDX_HALF, IDX_HALF)],
          sidx_v)
      pltpu.sync_copy(
          dst_hbm.at[pl.ds(w * CHUNKS_PER_TILE + h * IDX_HALF, IDX_HALF)],
          didx_v)
      gather(0, 0).start()

      def body(k, _):
        b = lax.rem(k, 2)
        gather(k, b).wait()
        scatter(k, b)

        @pl.when(k + 1 < IDX_HALF)
        def _():
          @pl.when(k >= 1)
          def _():
            scatter_wait(k - 1, 1 - b)
          gather(k + 1, 1 - b).start()
        return 0
      lax.fori_loop(0, IDX_HALF, body, 0)
      scatter_wait(IDX_HALF - 2, IDX_HALF % 2)
      scatter_wait(IDX_HALF - 1, (IDX_HALF - 1) % 2)
    plsc.subcore_barrier()

    pltpu.sync_copy(
        acc.at[pl.ds(s * ROWS_PER_TILE, ROWS_PER_TILE)],
        out_hbm.at[c, pl.ds(s * ROWS_PER_TILE, ROWS_PER_TILE)],
    )

  return agg_kernel(x, src_p, dst_p)


BLK = 1264


def _mm_body(p_ref, w_ref, b_ref, o_ref):
  s = p_ref[0] + p_ref[1]
  o_ref[...] = (
      jnp.dot(s, w_ref[...], preferred_element_type=jnp.float32) + b_ref[...]
  )


def _tc_matmul(parts, W, b2d):
  return pl.pallas_call(
      _mm_body,
      grid=(N_PAD // BLK,),
      in_specs=[
          pl.BlockSpec((NC, BLK, D), lambda i: (0, i, 0)),
          pl.BlockSpec((D, D), lambda i: (0, 0)),
          pl.BlockSpec((1, D), lambda i: (0, 0)),
      ],
      out_specs=pl.BlockSpec((BLK, D), lambda i: (i, 0)),
      out_shape=jax.ShapeDtypeStruct((N_PAD, D), jnp.float32),
  )(parts, W, b2d)


@jax.jit
def kernel(x, edge_index, W, b):
  src = edge_index[0].astype(jnp.int32)
  dst = edge_index[1].astype(jnp.int32)
  ntiles = NC * NS
  per_tile = N_EDGES // ntiles
  pad = E_TILE - per_tile
  pad_src = jnp.zeros((ntiles, pad), jnp.int32)
  pad_dst = jnp.broadcast_to(
      N_NODES + (jnp.arange(pad, dtype=jnp.int32) % (N_PAD - N_NODES)),
      (ntiles, pad))
  src_p = jnp.concatenate([src.reshape(ntiles, per_tile), pad_src], axis=1)
  dst_p = jnp.concatenate([dst.reshape(ntiles, per_tile), pad_dst], axis=1)
  src_p = src_p.reshape(E_PAD // CHUNK, CHUNK)
  dst_p = dst_p.reshape(E_PAD // CHUNK, CHUNK)
  parts = _sc_aggregate(x, src_p, dst_p)
  out = _tc_matmul(parts, W, b.reshape(1, D))
  return out[:N_NODES]


# speedup vs baseline: 1.0497x; 1.0497x over previous
import functools

import jax
import jax.numpy as jnp
from jax import lax
from jax.experimental import pallas as pl
from jax.experimental.pallas import tpu as pltpu
from jax.experimental.pallas import tpu_sc as plsc

N_NODES = 10000
N_EDGES = 320000
D = 128

NC = 2
NS = 16
CHUNK = 128
CHUNKS_PER_TILE = 80
IDX_HALF = 40
NBUF = 2
E_TILE = CHUNK * CHUNKS_PER_TILE
E_PAD = E_TILE * NC * NS
N_PAD = 10112
ROWS_PER_TILE = N_PAD // NS


def _sc_aggregate(x, src_p, dst_p):
  mesh = plsc.VectorSubcoreMesh(core_axis_name="c", subcore_axis_name="s")

  @functools.partial(
      pl.kernel,
      mesh=mesh,
      out_type=jax.ShapeDtypeStruct((NC, N_PAD, D), jnp.float32),
      scratch_types=[
          pltpu.VMEM((IDX_HALF, CHUNK), jnp.int32),
          pltpu.VMEM((IDX_HALF, CHUNK), jnp.int32),
          pltpu.VMEM((NBUF, CHUNK, D), jnp.float32),
          pltpu.VMEM_SHARED((N_PAD, D), jnp.float32),
          pltpu.SemaphoreType.DMA,
      ],
  )
  def agg_kernel(x_hbm, src_hbm, dst_hbm, out_hbm, sidx_v, didx_v, rows_v, acc,
                 gsem):
    c = lax.axis_index("c")
    s = lax.axis_index("s")
    w = s * NC + c

    def zero_row(i, _):
      for j in range(D // 16):
        rows_v[0, i, pl.ds(j * 16, 16)] = jnp.zeros((16,), jnp.float32)
      return 0
    lax.fori_loop(0, CHUNK, zero_row, 0)
    full = ROWS_PER_TILE // CHUNK
    for k in range(full):
      pltpu.sync_copy(rows_v.at[0],
                      acc.at[pl.ds(s * ROWS_PER_TILE + k * CHUNK, CHUNK)])
    rem = ROWS_PER_TILE - full * CHUNK
    if rem:
      pltpu.sync_copy(
          rows_v.at[0, pl.ds(0, rem)],
          acc.at[pl.ds(s * ROWS_PER_TILE + full * CHUNK, rem)])
    plsc.subcore_barrier()

    def gather(j, b):
      return pltpu.make_async_copy(x_hbm.at[sidx_v.at[j]], rows_v.at[b], gsem)

    for h in range(CHUNKS_PER_TILE // IDX_HALF):
      pltpu.sync_copy(
          src_hbm.at[pl.ds(w * CHUNKS_PER_TILE + h * IDX_HALF, IDX_HALF)],
          sidx_v)
      pltpu.sync_copy(
          dst_hbm.at[pl.ds(w * CHUNKS_PER_TILE + h * IDX_HALF, IDX_HALF)],
          didx_v)
      for b in range(NBUF):
        gather(b, b).start()

      def outer(j0, _):
        for b in range(NBUF):
          j = j0 + b
          gather(j, b).wait()
          pltpu.sync_copy(rows_v.at[b], acc.at[didx_v.at[j]], add=True)

          @pl.when(j + NBUF < IDX_HALF)
          def _():
            gather(j + NBUF, b).start()
        return 0
      lax.fori_loop(0, IDX_HALF // NBUF, lambda i, u: outer(i * NBUF, u), 0)
    plsc.subcore_barrier()

    pltpu.sync_copy(
        acc.at[pl.ds(s * ROWS_PER_TILE, ROWS_PER_TILE)],
        out_hbm.at[c, pl.ds(s * ROWS_PER_TILE, ROWS_PER_TILE)],
    )

  return agg_kernel(x, src_p, dst_p)


BLK = 1264


def _mm_body(p_ref, w_ref, b_ref, o_ref):
  s = p_ref[0] + p_ref[1]
  o_ref[...] = (
      jnp.dot(s, w_ref[...], preferred_element_type=jnp.float32) + b_ref[...]
  )


def _tc_matmul(parts, W, b2d):
  return pl.pallas_call(
      _mm_body,
      grid=(N_PAD // BLK,),
      in_specs=[
          pl.BlockSpec((NC, BLK, D), lambda i: (0, i, 0)),
          pl.BlockSpec((D, D), lambda i: (0, 0)),
          pl.BlockSpec((1, D), lambda i: (0, 0)),
      ],
      out_specs=pl.BlockSpec((BLK, D), lambda i: (i, 0)),
      out_shape=jax.ShapeDtypeStruct((N_PAD, D), jnp.float32),
  )(parts, W, b2d)


@jax.jit
def kernel(x, edge_index, W, b):
  src = edge_index[0].astype(jnp.int32)
  dst = edge_index[1].astype(jnp.int32)
  ntiles = NC * NS
  per_tile = N_EDGES // ntiles
  pad = E_TILE - per_tile
  pad_src = jnp.zeros((ntiles, pad), jnp.int32)
  pad_dst = jnp.broadcast_to(
      N_NODES + (jnp.arange(pad, dtype=jnp.int32) % (N_PAD - N_NODES)),
      (ntiles, pad))
  src_p = jnp.concatenate([src.reshape(ntiles, per_tile), pad_src], axis=1)
  dst_p = jnp.concatenate([dst.reshape(ntiles, per_tile), pad_dst], axis=1)
  src_p = src_p.reshape(E_PAD // CHUNK, CHUNK)
  dst_p = dst_p.reshape(E_PAD // CHUNK, CHUNK)
  parts = _sc_aggregate(x, src_p, dst_p)
  out = _tc_matmul(parts, W, b.reshape(1, D))
  return out[:N_NODES]
